# baseline (speedup 1.0000x reference)
import jax
import jax.numpy as jnp
from jax.experimental import pallas as pl
from jax.experimental.pallas import tpu as pltpu

_N = 10000
_F = 128
_FUSSY = 4
_BM = 400


def _body(mu1_ref, sig1_ref, mu2_ref, sig2_ref, x_ref, adj_ref, w1_ref,
          w2_ref, b1_ref, b2_ref, out_ref, y_ref, y2_ref):
    l = pl.program_id(0)
    i = pl.program_id(1)

    @pl.when(jnp.logical_and(l == 0, i == 0))
    def _init_y1():
        y_ref[...] = jnp.dot(x_ref[...], w1_ref[...],
                             preferred_element_type=jnp.float32)

    z = jax.lax.cond(
        l == 0,
        lambda: jnp.dot(adj_ref[...], y_ref[...],
                        preferred_element_type=jnp.float32),
        lambda: jnp.dot(adj_ref[...], y2_ref[...],
                        preferred_element_type=jnp.float32),
    )
    z = z + jnp.where(l == 0, b1_ref[...], b2_ref[...])
    acc = None
    for k in range(_FUSSY):
        m = jnp.where(l == 0, mu1_ref[k], mu2_ref[k])
        s = jnp.where(l == 0, sig1_ref[k], sig2_ref[k])
        d = z - m
        t = jnp.exp(d * d * (-1.0 / (s * s)))
        acc = t if acc is None else acc + t
    gated = z * (acc * (1.0 / _FUSSY))

    @pl.when(l == 0)
    def _store_layer1():
        y2_ref[pl.ds(i * _BM, _BM), :] = jnp.dot(
            gated, w2_ref[...], preferred_element_type=jnp.float32)

    @pl.when(l == 1)
    def _store_layer2():
        out_ref[...] = gated


def kernel(x, adj, W1, b1, mu1, sig1, W2, b2, mu2, sig2):
    return pl.pallas_call(
        _body,
        grid=(2, _N // _BM),
        in_specs=[
            pl.BlockSpec(memory_space=pltpu.SMEM),
            pl.BlockSpec(memory_space=pltpu.SMEM),
            pl.BlockSpec(memory_space=pltpu.SMEM),
            pl.BlockSpec(memory_space=pltpu.SMEM),
            pl.BlockSpec((_N, _F), lambda l, i: (0, 0)),
            pl.BlockSpec((_BM, _N), lambda l, i: (i, 0)),
            pl.BlockSpec((_F, _F), lambda l, i: (0, 0)),
            pl.BlockSpec((_F, _F), lambda l, i: (0, 0)),
            pl.BlockSpec((1, _F), lambda l, i: (0, 0)),
            pl.BlockSpec((1, _F), lambda l, i: (0, 0)),
        ],
        out_specs=pl.BlockSpec((_BM, _F), lambda l, i: (i * l, 0)),
        out_shape=jax.ShapeDtypeStruct((_N, _F), jnp.float32),
        scratch_shapes=[
            pltpu.VMEM((_N, _F), jnp.float32),
            pltpu.VMEM((_N, _F), jnp.float32),
        ],
        compiler_params=pltpu.CompilerParams(
            vmem_limit_bytes=100 * 1024 * 1024,
        ),
    )(mu1, sig1, mu2, sig2, x, adj, W1, W2,
      b1.reshape(1, _F), b2.reshape(1, _F))


# speedup vs baseline: 1.0053x; 1.0053x over previous
import jax
import jax.numpy as jnp
from jax.experimental import pallas as pl
from jax.experimental.pallas import tpu as pltpu

_N = 10000
_F = 128
_FUSSY = 4
_BM = 400


def _body(mu1_ref, sig1_ref, mu2_ref, sig2_ref, x_ref, adj_ref, w1_ref,
          w2_ref, b1_ref, b2_ref, out_ref, y_ref, y2_ref, stash_ref):
    l = pl.program_id(0)
    i = pl.program_id(1)
    nb = _N // _BM

    @pl.when(jnp.logical_and(l == 0, i == 0))
    def _init_y1():
        y_ref[...] = jnp.dot(x_ref[...], w1_ref[...],
                             preferred_element_type=jnp.float32)

    @pl.when(jnp.logical_and(l == 0, i == nb - 1))
    def _fill_stash():
        stash_ref[...] = adj_ref[...]

    z = jax.lax.cond(
        l == 0,
        lambda: jnp.dot(adj_ref[...], y_ref[...],
                        preferred_element_type=jnp.float32),
        lambda: jax.lax.cond(
            i == nb - 1,
            lambda: jnp.dot(stash_ref[...], y2_ref[...],
                            preferred_element_type=jnp.float32),
            lambda: jnp.dot(adj_ref[...], y2_ref[...],
                            preferred_element_type=jnp.float32),
        ),
    )
    z = z + jnp.where(l == 0, b1_ref[...], b2_ref[...])
    acc = None
    for k in range(_FUSSY):
        m = jnp.where(l == 0, mu1_ref[k], mu2_ref[k])
        s = jnp.where(l == 0, sig1_ref[k], sig2_ref[k])
        d = z - m
        t = jnp.exp(d * d * (-1.0 / (s * s)))
        acc = t if acc is None else acc + t
    gated = z * (acc * (1.0 / _FUSSY))

    @pl.when(l == 0)
    def _store_layer1():
        y2_ref[pl.ds(i * _BM, _BM), :] = jnp.dot(
            gated, w2_ref[...], preferred_element_type=jnp.float32)

    @pl.when(l == 1)
    def _store_layer2():
        out_ref[...] = gated


def kernel(x, adj, W1, b1, mu1, sig1, W2, b2, mu2, sig2):
    return pl.pallas_call(
        _body,
        grid=(2, _N // _BM),
        in_specs=[
            pl.BlockSpec(memory_space=pltpu.SMEM),
            pl.BlockSpec(memory_space=pltpu.SMEM),
            pl.BlockSpec(memory_space=pltpu.SMEM),
            pl.BlockSpec(memory_space=pltpu.SMEM),
            pl.BlockSpec((_N, _F), lambda l, i: (0, 0)),
            pl.BlockSpec((_BM, _N),
                         lambda l, i: (jnp.minimum(i, _N // _BM - 1 - l), 0)),
            pl.BlockSpec((_F, _F), lambda l, i: (0, 0)),
            pl.BlockSpec((_F, _F), lambda l, i: (0, 0)),
            pl.BlockSpec((1, _F), lambda l, i: (0, 0)),
            pl.BlockSpec((1, _F), lambda l, i: (0, 0)),
        ],
        out_specs=pl.BlockSpec((_BM, _F), lambda l, i: (i * l, 0)),
        out_shape=jax.ShapeDtypeStruct((_N, _F), jnp.float32),
        scratch_shapes=[
            pltpu.VMEM((_N, _F), jnp.float32),
            pltpu.VMEM((_N, _F), jnp.float32),
            pltpu.VMEM((_BM, _N), jnp.float32),
        ],
        compiler_params=pltpu.CompilerParams(
            vmem_limit_bytes=100 * 1024 * 1024,
        ),
    )(mu1, sig1, mu2, sig2, x, adj, W1, W2,
      b1.reshape(1, _F), b2.reshape(1, _F))
